# Initial kernel scaffold; baseline (speedup 1.0000x reference)
import jax
import jax.numpy as jnp
from jax.experimental import pallas as pl


def kernel(ent_embed, rel_embed, edge_index, edge_type, batch_idx, W, W_loop, W_rel, b):
    raise NotImplementedError("write your pallas kernel here")


# speedup vs baseline: 4.0899x; 4.0899x over previous
import functools

import jax
import jax.numpy as jnp
from jax import lax
from jax.experimental import pallas as pl
from jax.experimental.pallas import tpu as pltpu
from jax.experimental.pallas import tpu_sc as plsc

N = 10000
E = 320000
D = 128
NREL = 200
Q = 500
NC, NS = 2, 16
K = 80
RCH = N // K
ZCH_PER_TILE = (RCH + NS - 1) // NS

@functools.cache
def _mesh():
    return plsc.VectorSubcoreMesh(core_axis_name="c", subcore_axis_name="s",
                                  num_cores=NC, num_subcores=NS)


def _fill_const(ref, nrow, ncol, val):
    @pl.loop(0, nrow)
    def _(r):
        @pl.loop(0, ncol // 16)
        def _(c):
            ref[r, pl.ds(c * 16, 16)] = jnp.full((16,), val, jnp.float32)


def _zero_acc_chunks(sid, zv, zdv, acc_s, acc_d):
    @pl.loop(0, ZCH_PER_TILE)
    def _(k):
        ch = sid + k * NS
        @pl.when(ch < RCH)
        def _():
            pltpu.sync_copy(zv, acc_s.at[pl.ds(ch * K, K)])
            if acc_d is not None:
                pltpu.sync_copy(zdv, acc_d.at[pl.ds(ch * K, K)])


def _edge_loop(table, idx_hbm, dst_hbm, ebase, nchunks, acc, idxg, idxs, rows,
               sem, onesv=None, acc_d=None):
    @pl.loop(0, nchunks)
    def _(i):
        base = ebase + i * K
        pltpu.sync_copy(idx_hbm.at[pl.ds(base, K)], idxg)
        pltpu.sync_copy(dst_hbm.at[pl.ds(base, K)], idxs)
        pltpu.async_copy(table.at[idxg], rows, sem).wait()
        pltpu.sync_copy(rows, acc.at[idxs], add=True)
        if onesv is not None:
            pltpu.sync_copy(onesv, acc_d.at[idxs], add=True)


def _sc_deg_body(dst, dega, degb, acc_d, idxs, onesv, zdv):
    cid = lax.axis_index("c")
    sid = lax.axis_index("s")
    _fill_const(zdv, K, D, 0.0)
    _fill_const(onesv, K, D, 1.0)
    @pl.loop(0, ZCH_PER_TILE)
    def _(k):
        ch = sid + k * NS
        @pl.when(ch < RCH)
        def _():
            pltpu.sync_copy(zdv, acc_d.at[pl.ds(ch * K, K)])
    plsc.subcore_barrier()
    ebase = cid * (E // NC) + sid * (E // NC // NS)
    @pl.loop(0, E // NC // NS // K)
    def _(i):
        pltpu.sync_copy(dst.at[pl.ds(ebase + i * K, K)], idxs)
        pltpu.sync_copy(onesv, acc_d.at[idxs], add=True)
    plsc.subcore_barrier()
    @pl.loop(0, ZCH_PER_TILE)
    def _(k):
        ch = sid + k * NS
        @pl.when(ch < RCH)
        def _():
            r0 = ch * K
            pltpu.sync_copy(acc_d.at[pl.ds(r0, K)], zdv)
            @pl.when(cid == 0)
            def _():
                pltpu.sync_copy(zdv, dega.at[pl.ds(r0, K)])
            @pl.when(cid == 1)
            def _():
                pltpu.sync_copy(zdv, degb.at[pl.ds(r0, K)])


@functools.cache
def _sc_deg():
  return pl.kernel(
    _sc_deg_body,
    out_type=(
        jax.ShapeDtypeStruct((N, D), jnp.float32),
        jax.ShapeDtypeStruct((N, D), jnp.float32),
    ),
    mesh=_mesh(),
    scratch_types=[
        pltpu.VMEM_SHARED((N, D), jnp.float32),
        pltpu.VMEM((K,), jnp.int32),
        pltpu.VMEM((K, D), jnp.float32),
        pltpu.VMEM((K, D), jnp.float32),
    ],
)


def _sc_pass1_body(ent, rel, src, dst, et,
                   g1, cr, acc_s, idxg, idxs, rows, zv, sem):
    cid = lax.axis_index("c")
    sid = lax.axis_index("s")
    _fill_const(zv, K, D, 0.0)
    _zero_acc_chunks(sid, zv, None, acc_s, None)
    plsc.subcore_barrier()
    ebase = sid * (E // NS)
    @pl.when(cid == 0)
    def _():
        _edge_loop(ent, src, dst, ebase, E // NS // K, acc_s, idxg, idxs,
                   rows, sem)
    @pl.when(cid == 1)
    def _():
        _edge_loop(rel, et, dst, ebase, E // NS // K, acc_s, idxg, idxs,
                   rows, sem)
    plsc.subcore_barrier()
    @pl.loop(0, ZCH_PER_TILE)
    def _(k):
        ch = sid + k * NS
        @pl.when(ch < RCH)
        def _():
            r0 = ch * K
            pltpu.sync_copy(acc_s.at[pl.ds(r0, K)], rows)
            @pl.when(cid == 0)
            def _():
                pltpu.sync_copy(rows, g1.at[pl.ds(r0, K)])
            @pl.when(cid == 1)
            def _():
                pltpu.sync_copy(rows, cr.at[pl.ds(r0, K)])


@functools.cache
def _sc_pass1():
  return pl.kernel(
    _sc_pass1_body,
    out_type=(
        jax.ShapeDtypeStruct((N, D), jnp.float32),
        jax.ShapeDtypeStruct((N, D), jnp.float32),
    ),
    mesh=_mesh(),
    scratch_types=[
        pltpu.VMEM_SHARED((N, D), jnp.float32),
        pltpu.VMEM((K,), jnp.int32),
        pltpu.VMEM((K,), jnp.int32),
        pltpu.VMEM((K, D), jnp.float32),
        pltpu.VMEM((K, D), jnp.float32),
        pltpu.SemaphoreType.DMA,
    ],
)


def _sc_pass2_body(h1, src, dst, g2a, g2b, acc_s, idxg, idxs, rows, zv,
                   sem):
    cid = lax.axis_index("c")
    sid = lax.axis_index("s")
    _fill_const(zv, K, D, 0.0)
    _zero_acc_chunks(sid, zv, None, acc_s, None)
    plsc.subcore_barrier()
    ebase = cid * (E // NC) + sid * (E // NC // NS)
    _edge_loop(h1, src, dst, ebase, E // NC // NS // K, acc_s, idxg, idxs,
               rows, sem)
    plsc.subcore_barrier()
    @pl.loop(0, ZCH_PER_TILE)
    def _(k):
        ch = sid + k * NS
        @pl.when(ch < RCH)
        def _():
            r0 = ch * K
            pltpu.sync_copy(acc_s.at[pl.ds(r0, K)], rows)
            @pl.when(cid == 0)
            def _():
                pltpu.sync_copy(rows, g2a.at[pl.ds(r0, K)])
            @pl.when(cid == 1)
            def _():
                pltpu.sync_copy(rows, g2b.at[pl.ds(r0, K)])


@functools.cache
def _sc_pass2():
  return pl.kernel(
    _sc_pass2_body,
    out_type=(
        jax.ShapeDtypeStruct((N, D), jnp.float32),
        jax.ShapeDtypeStruct((N, D), jnp.float32),
    ),
    mesh=_mesh(),
    scratch_types=[
        pltpu.VMEM_SHARED((N, D), jnp.float32),
        pltpu.VMEM((K,), jnp.int32),
        pltpu.VMEM((K,), jnp.int32),
        pltpu.VMEM((K, D), jnp.float32),
        pltpu.VMEM((K, D), jnp.float32),
        pltpu.SemaphoreType.DMA,
    ],
)


QP = 512


def _sc_readout_body(x2, bidx, out, acc_q, idxs, rows, zqv):
    cid = lax.axis_index("c")
    sid = lax.axis_index("s")
    @pl.when((cid == 0) & (sid < 4))
    def _():
        _fill_const(zqv, 128, D, 0.0)
        pltpu.sync_copy(zqv, acc_q.at[pl.ds(sid * 128, 128)])
    plsc.subcore_barrier()
    @pl.when(cid == 0)
    def _():
        @pl.loop(0, ZCH_PER_TILE)
        def _(k):
            ch = sid + k * NS
            @pl.when(ch < RCH)
            def _():
                base = ch * K
                pltpu.sync_copy(x2.at[pl.ds(base, K)], rows)
                pltpu.sync_copy(bidx.at[pl.ds(base, K)], idxs)
                pltpu.sync_copy(rows, acc_q.at[idxs], add=True)
    plsc.subcore_barrier()
    @pl.when((cid == 0) & (sid < 4))
    def _():
        q0 = sid * 128
        pltpu.sync_copy(acc_q.at[pl.ds(q0, 128)], zqv)
        pltpu.sync_copy(zqv, out.at[pl.ds(q0, 128)])


@functools.cache
def _sc_readout():
  return pl.kernel(
    _sc_readout_body,
    out_type=jax.ShapeDtypeStruct((QP, D), jnp.float32),
    mesh=_mesh(),
    scratch_types=[
        pltpu.VMEM_SHARED((QP, D), jnp.float32),
        pltpu.VMEM((K,), jnp.int32),
        pltpu.VMEM((K, D), jnp.float32),
        pltpu.VMEM((128, D), jnp.float32),
    ],
)


BM = 1000


def _tc1_body(g1, cr, dga, dgb, x, w, wl, bb, o):
    invd = 1.0 / jnp.maximum(dga[:, 0:1] + dgb[:, 0:1], 1.0)
    pre = (g1[...] - cr[...]) * invd
    acc = jnp.dot(pre, w[...], preferred_element_type=jnp.float32)
    acc += jnp.dot(x[...], wl[...], preferred_element_type=jnp.float32)
    o[...] = jnp.maximum(acc + bb[...], 0.0)


def _tc2_body(g2a, g2b, cr, dga, dgb, h1, w, wr, wl, bb, o):
    invd = 1.0 / jnp.maximum(dga[:, 0:1] + dgb[:, 0:1], 1.0)
    crw = jnp.dot(cr[...], wr[...], preferred_element_type=jnp.float32)
    pre = (g2a[...] + g2b[...] - crw) * invd
    acc = jnp.dot(pre, w[...], preferred_element_type=jnp.float32)
    acc += jnp.dot(h1[...], wl[...], preferred_element_type=jnp.float32)
    o[...] = acc + bb[...]


def _row_spec(cols):
    return pl.BlockSpec((BM, cols), lambda i: (i, 0))


def _full_spec(r, c):
    return pl.BlockSpec((r, c), lambda i: (0, 0))


_tc1 = pl.pallas_call(
    _tc1_body,
    grid=(N // BM,),
    in_specs=[_row_spec(D), _row_spec(D), _row_spec(D), _row_spec(D),
              _row_spec(D), _full_spec(D, D), _full_spec(D, D),
              _full_spec(1, D)],
    out_specs=_row_spec(D),
    out_shape=jax.ShapeDtypeStruct((N, D), jnp.float32),
)

_tc2 = pl.pallas_call(
    _tc2_body,
    grid=(N // BM,),
    in_specs=[_row_spec(D), _row_spec(D), _row_spec(D), _row_spec(D),
              _row_spec(D), _row_spec(D), _full_spec(D, D),
              _full_spec(D, D), _full_spec(D, D), _full_spec(1, D)],
    out_specs=_row_spec(D),
    out_shape=jax.ShapeDtypeStruct((N, D), jnp.float32),
)


def kernel(ent_embed, rel_embed, edge_index, edge_type, batch_idx, W, W_loop,
           W_rel, b):
    src = edge_index[0]
    dst = edge_index[1]
    b2 = b.reshape(1, D)

    dega, degb = _sc_deg()(dst)
    g1, cr = _sc_pass1()(ent_embed, rel_embed, src, dst, edge_type)
    h1 = _tc1(g1, cr, dega, degb, ent_embed, W, W_loop, b2)
    g2a, g2b = _sc_pass2()(h1, src, dst)
    x2 = _tc2(g2a, g2b, cr, dega, degb, h1, W, W_rel, W_loop, b2)
    return _sc_readout()(x2, batch_idx)[:Q]
